# Initial kernel scaffold; baseline (speedup 1.0000x reference)
import numpy as np
import jax
import jax.numpy as jnp
from jax import lax
from jax.experimental import pallas as pl
from jax.experimental.pallas import tpu as pltpu
from jax.experimental.pallas import tpu_sc as plsc

D = 64
SEQ = 50
ROWS = 4096 * SEQ
NC, NS = 2, 16
NW = NC * NS
RPW = ROWS // NW
CHUNK = 128
NCHUNK = RPW // CHUNK
SCALE = 8.0
POS_TILE = 200


def _pos_const():
    depth = D / 2
    positions = np.arange(SEQ)[:, None]
    depths = np.arange(depth)[None, :] / depth
    angle_rates = 1 / 10000 ** depths
    angle_rads = positions * angle_rates
    pos = np.concatenate([np.sin(angle_rads), np.cos(angle_rads)], axis=-1)
    pos = np.concatenate([pos] * (POS_TILE // SEQ), axis=0)
    return jnp.asarray(pos, dtype=jnp.float32)


def _body(x_hbm, table_hbm, pos_hbm, out_hbm, idx_v, pos_v, rows_v, sem):
    wid = lax.axis_index("s") * NC + lax.axis_index("c")
    base = wid * RPW
    pltpu.sync_copy(pos_hbm, pos_v)
    pltpu.sync_copy(x_hbm.at[pl.ds(base, RPW)], idx_v)

    def chunk_body(k, carry):
        row0 = k * CHUNK
        p0 = lax.rem(row0, SEQ)
        pltpu.async_copy(
            table_hbm.at[idx_v.at[pl.ds(row0, CHUNK)]], rows_v, sem
        ).wait()

        def row_body(r, c):
            p = p0 + r
            for j in range(D // 16):
                sl = pl.ds(j * 16, 16)
                rows_v[r, sl] = rows_v[r, sl] * SCALE + pos_v[p, sl]
            return c

        lax.fori_loop(0, CHUNK, row_body, 0)
        pltpu.sync_copy(rows_v, out_hbm.at[pl.ds(base + row0, CHUNK)])
        return carry

    lax.fori_loop(0, NCHUNK, chunk_body, 0)


def kernel(x, table):
    mesh = plsc.VectorSubcoreMesh(core_axis_name="c", subcore_axis_name="s")
    f = pl.kernel(
        _body,
        out_type=jax.ShapeDtypeStruct((ROWS, D), jnp.float32),
        mesh=mesh,
        scratch_types=[
            pltpu.VMEM((RPW,), jnp.int32),
            pltpu.VMEM((POS_TILE, D), jnp.float32),
            pltpu.VMEM((CHUNK, D), jnp.float32),
            pltpu.SemaphoreType.DMA,
        ],
    )
    out = f(x.reshape(-1).astype(jnp.int32), table, _pos_const())
    return out.reshape(x.shape[0], x.shape[1], D)


# speedup vs baseline: 2.7734x; 2.7734x over previous
import numpy as np
import jax
import jax.numpy as jnp
from jax import lax
from jax.experimental import pallas as pl
from jax.experimental.pallas import tpu as pltpu
from jax.experimental.pallas import tpu_sc as plsc

D = 64
SEQ = 50
ROWS = 4096 * SEQ
NC, NS = 2, 16
NW = NC * NS
RPW = ROWS // NW
CHUNK = 128
NCHUNK = RPW // CHUNK
SCALE = 8.0
POS_TILE = 200


def _pos_const():
    depth = D / 2
    positions = np.arange(SEQ)[:, None]
    depths = np.arange(depth)[None, :] / depth
    angle_rates = 1 / 10000 ** depths
    angle_rads = positions * angle_rates
    pos = np.concatenate([np.sin(angle_rads), np.cos(angle_rads)], axis=-1)
    pos = np.concatenate([pos] * (POS_TILE // SEQ), axis=0)
    return jnp.asarray(pos, dtype=jnp.float32)


def _body(x_hbm, table_hbm, pos_hbm, out_hbm, idx_v, pos_v, rows_v, sem):
    wid = lax.axis_index("s") * NC + lax.axis_index("c")
    base = wid * RPW
    pltpu.sync_copy(pos_hbm, pos_v)
    pltpu.sync_copy(x_hbm.at[pl.ds(base, RPW)], idx_v)

    def chunk_body(k, carry):
        row0 = k * CHUNK
        p0 = lax.rem(row0, SEQ)
        pltpu.async_copy(
            table_hbm.at[idx_v.at[pl.ds(row0, CHUNK)]], rows_v, sem
        ).wait()

        def row_body(r, c):
            p = p0 + r
            for j in range(D // 16):
                sl = pl.ds(j * 16, 16)
                rows_v[r, sl] = rows_v[r, sl] * SCALE + pos_v[p, sl]
            return c

        lax.fori_loop(0, CHUNK, row_body, 0)
        pltpu.sync_copy(rows_v, out_hbm.at[pl.ds(base + row0, CHUNK)])
        return carry

    lax.fori_loop(0, NCHUNK, chunk_body, 0)


def kernel(x, table):
    mesh = plsc.VectorSubcoreMesh(core_axis_name="c", subcore_axis_name="s")
    f = pl.kernel(
        _body,
        out_type=jax.ShapeDtypeStruct((ROWS, D), jnp.float32),
        mesh=mesh,
        scratch_types=[
            pltpu.VMEM((RPW,), jnp.int32),
            pltpu.VMEM((POS_TILE, D), jnp.float32),
            pltpu.VMEM((CHUNK, D), jnp.float32),
            pltpu.SemaphoreType.DMA,
        ],
        compiler_params=pltpu.CompilerParams(use_tc_tiling_on_sc=False),
    )
    out = f(x.reshape(-1).astype(jnp.int32), table, _pos_const())
    return out.reshape(x.shape[0], x.shape[1], D)


# speedup vs baseline: 4.5961x; 1.6572x over previous
import numpy as np
import jax
import jax.numpy as jnp
from jax import lax
from jax.experimental import pallas as pl
from jax.experimental.pallas import tpu as pltpu
from jax.experimental.pallas import tpu_sc as plsc

D = 64
SEQ = 50
ROWS = 4096 * SEQ
NC, NS = 2, 16
NW = NC * NS
RPW = ROWS // NW
CHUNK = 200
G1, G2 = 128, 72
NCHUNK = RPW // CHUNK
NBUF = 4
SCALE = 8.0


def _pos_const():
    depth = D / 2
    positions = np.arange(SEQ)[:, None]
    depths = np.arange(depth)[None, :] / depth
    angle_rates = 1 / 10000 ** depths
    angle_rads = positions * angle_rates
    pos = np.concatenate([np.sin(angle_rads), np.cos(angle_rads)], axis=-1)
    return jnp.asarray(pos, dtype=jnp.float32)


def _body(x_hbm, table_hbm, pos_hbm, out_hbm, idx_v, pos_v, bufs,
          gs0, gs1, gs2, gs3, ss0, ss1, ss2, ss3):
    gsems = (gs0, gs1, gs2, gs3)
    ssems = (ss0, ss1, ss2, ss3)
    wid = lax.axis_index("s") * NC + lax.axis_index("c")
    base = wid * RPW
    pltpu.sync_copy(pos_hbm, pos_v)
    pltpu.sync_copy(x_hbm.at[pl.ds(base, RPW)], idx_v)

    def _gather_parts(k, b):
        row0 = k * CHUNK
        yield (table_hbm.at[idx_v.at[pl.ds(row0, G1)]],
               bufs.at[b].at[pl.ds(0, G1)], gsems[b])
        yield (table_hbm.at[idx_v.at[pl.ds(row0 + G1, G2)]],
               bufs.at[b].at[pl.ds(G1, G2)], gsems[b])

    def fire_gather(k, b):
        for src, dst, sem in _gather_parts(k, b):
            pltpu.async_copy(src, dst, sem)

    def wait_gather(k, b):
        for src, dst, sem in _gather_parts(k, b):
            pltpu.make_async_copy(src, dst, sem).wait()

    def fire_store(k, b):
        pltpu.async_copy(bufs.at[b],
                         out_hbm.at[pl.ds(base + k * CHUNK, CHUNK)], ssems[b])

    def wait_store(k, b):
        pltpu.make_async_copy(bufs.at[b],
                              out_hbm.at[pl.ds(base + k * CHUNK, CHUNK)],
                              ssems[b]).wait()

    def compute(b):
        @plsc.parallel_loop(0, SEQ, unroll=2)
        def _(q):
            for j in range(D // 16):
                sl = pl.ds(j * 16, 16)
                pv = pos_v[q, sl]
                for r0 in range(0, CHUNK, SEQ):
                    bufs[b, q + r0, sl] = bufs[b, q + r0, sl] * SCALE + pv

    for b in range(NBUF - 1):
        fire_gather(b, b)
    for b in range(NBUF):
        if b == 0:
            fire_gather(NBUF - 1, NBUF - 1)
        else:
            wait_store(b - 1, b - 1)
            fire_gather(b + NBUF - 1, b - 1)
        wait_gather(b, b)
        compute(b)
        fire_store(b, b)

    def block(m, carry):
        k0 = m * NBUF
        for b in range(NBUF):
            k = k0 + b
            bp = (b + NBUF - 1) % NBUF
            wait_store(k - 1, bp)
            fire_gather(k + NBUF - 1, bp)
            wait_gather(k, b)
            compute(b)
            fire_store(k, b)
        return carry

    lax.fori_loop(1, NCHUNK // NBUF - 1, block, 0)

    k0 = NCHUNK - NBUF
    for b in range(NBUF):
        k = k0 + b
        if b == 0:
            wait_store(k - 1, NBUF - 1)
            fire_gather(k + NBUF - 1, NBUF - 1)
        wait_gather(k, b)
        compute(b)
        fire_store(k, b)
    for b in range(NBUF):
        wait_store(k0 + b, b)


def kernel(x, table):
    mesh = plsc.VectorSubcoreMesh(core_axis_name="c", subcore_axis_name="s")
    f = pl.kernel(
        _body,
        out_type=jax.ShapeDtypeStruct((ROWS, D), jnp.float32),
        mesh=mesh,
        scratch_types=[
            pltpu.VMEM((RPW,), jnp.int32),
            pltpu.VMEM((SEQ, D), jnp.float32),
            pltpu.VMEM((NBUF, CHUNK, D), jnp.float32),
        ] + [pltpu.SemaphoreType.DMA] * (2 * NBUF),
        compiler_params=pltpu.CompilerParams(use_tc_tiling_on_sc=False),
    )
    out = f(x.reshape(-1).astype(jnp.int32), table, _pos_const())
    return out.reshape(x.shape[0], x.shape[1], D)
